# baseline (speedup 1.0000x reference)
import functools

import jax
import jax.numpy as jnp
from jax.experimental import pallas as pl
from jax.experimental.pallas import tpu as pltpu
from jax.sharding import Mesh, PartitionSpec as P
from jax.experimental.shard_map import shard_map

_TM = 256
_WCHUNK = 256
_NSTAGE = 2


def _linear_kernel(w_hbm, x_ref, b_ref, o_ref, wb_ref, stage_ref, sem):
    t = pl.program_id(0)
    nrows = wb_ref.shape[0]
    nchunks = nrows // _WCHUNK

    def copy(c, buf):
        return pltpu.make_async_copy(
            w_hbm.at[pl.ds(c * _WCHUNK, _WCHUNK), :],
            stage_ref.at[buf],
            sem.at[buf],
        )

    @pl.when(t == 0)
    def _():
        copy(0, 0).start()
        for c in range(nchunks):
            if c + 1 < nchunks:
                copy(c + 1, (c + 1) % 2).start()
            copy(c, c % 2).wait()
            wb_ref[pl.ds(c * _WCHUNK, _WCHUNK), :] = (
                stage_ref[c % 2].astype(jnp.bfloat16))

    xb = x_ref[...].astype(jnp.bfloat16)
    o_ref[...] = jax.lax.dot_general(
        xb, wb_ref[...], (((1,), (1,)), ((), ())),
        preferred_element_type=jnp.float32) + b_ref[...]


def _linear_local(x2d, weight, b2):
    M, K = x2d.shape
    N = weight.shape[0]
    grid = (M // _TM,)
    return pl.pallas_call(
        _linear_kernel,
        out_shape=jax.ShapeDtypeStruct((M, N), jnp.float32),
        grid=grid,
        in_specs=[
            pl.BlockSpec(memory_space=pl.ANY),
            pl.BlockSpec((_TM, K), lambda t: (t, 0)),
            pl.BlockSpec((1, N), lambda t: (0, 0)),
        ],
        out_specs=pl.BlockSpec((_TM, N), lambda t: (t, 0)),
        scratch_shapes=[
            pltpu.VMEM((N, K), jnp.bfloat16),
            pltpu.VMEM((_NSTAGE, _WCHUNK, K), jnp.float32),
            pltpu.SemaphoreType.DMA((_NSTAGE,)),
        ],
        compiler_params=pltpu.CompilerParams(
            dimension_semantics=("arbitrary",),
            vmem_limit_bytes=100 * 1024 * 1024,
        ),
        cost_estimate=pl.CostEstimate(
            flops=2 * M * N * K,
            transcendentals=0,
            bytes_accessed=(M * K + N * K + M * N) * 4,
        ),
    )(weight, x2d, b2)


@functools.partial(jax.jit, static_argnames=())
def _linear_sharded(x2d, weight, b2):
    devs = jax.devices()[:2]
    mesh = Mesh(devs, ("d",))
    fn = shard_map(
        _linear_local,
        mesh=mesh,
        in_specs=(P(None, None), P("d", None), P(None, "d")),
        out_specs=P(None, "d"),
        check_rep=False,
    )
    return fn(x2d, weight, b2)


def kernel(x, weight, bias):
    orig_shape = x.shape
    K = orig_shape[-1]
    N = weight.shape[0]
    x2d = x.reshape(-1, K)
    out = _linear_sharded(x2d, weight, bias.reshape(1, N))
    return out.reshape(*orig_shape[:-1], N)


# speedup vs baseline: 3.2754x; 3.2754x over previous
import functools

import jax
import jax.numpy as jnp
from jax.experimental import pallas as pl
from jax.experimental.pallas import tpu as pltpu

_TM = 256
_WCHUNK = 256
_NSTAGE = 2


def _linear_kernel(w_hbm, x_ref, b_ref, o_ref, wb_ref, stage_ref, sem):
    t = pl.program_id(0)
    N = wb_ref.shape[0]
    nchunks = N // _WCHUNK

    xb = x_ref[...].astype(jnp.bfloat16)

    def copy(c, buf):
        return pltpu.make_async_copy(
            w_hbm.at[pl.ds(c * _WCHUNK, _WCHUNK), :],
            stage_ref.at[buf],
            sem.at[buf],
        )

    @pl.when(t == 0)
    def _():
        copy(0, 0).start()
        for c in range(nchunks):
            if c + 1 < nchunks:
                copy(c + 1, (c + 1) % 2).start()
            copy(c, c % 2).wait()
            wc = stage_ref[c % 2].astype(jnp.bfloat16)
            wb_ref[pl.ds(c * _WCHUNK, _WCHUNK), :] = wc
            cols = pl.ds(c * _WCHUNK, _WCHUNK)
            o_ref[:, cols] = jax.lax.dot_general(
                xb, wc, (((1,), (1,)), ((), ())),
                preferred_element_type=jnp.float32) + b_ref[:, cols]

    @pl.when(t != 0)
    def _():
        o_ref[...] = jax.lax.dot_general(
            xb, wb_ref[...], (((1,), (1,)), ((), ())),
            preferred_element_type=jnp.float32) + b_ref[...]


@functools.partial(jax.jit, static_argnames=())
def _linear(x2d, weight, b2):
    M, K = x2d.shape
    N = weight.shape[0]
    grid = (M // _TM,)
    return pl.pallas_call(
        _linear_kernel,
        out_shape=jax.ShapeDtypeStruct((M, N), jnp.float32),
        grid=grid,
        in_specs=[
            pl.BlockSpec(memory_space=pl.ANY),
            pl.BlockSpec((_TM, K), lambda t: (t, 0)),
            pl.BlockSpec((1, N), lambda t: (0, 0)),
        ],
        out_specs=pl.BlockSpec((_TM, N), lambda t: (t, 0)),
        scratch_shapes=[
            pltpu.VMEM((N, K), jnp.bfloat16),
            pltpu.VMEM((_NSTAGE, _WCHUNK, K), jnp.float32),
            pltpu.SemaphoreType.DMA((_NSTAGE,)),
        ],
        compiler_params=pltpu.CompilerParams(
            dimension_semantics=("arbitrary",),
            vmem_limit_bytes=100 * 1024 * 1024,
        ),
        cost_estimate=pl.CostEstimate(
            flops=2 * M * N * K,
            transcendentals=0,
            bytes_accessed=(M * K + N * K + M * N) * 4,
        ),
    )(weight, x2d, b2)


def kernel(x, weight, bias):
    orig_shape = x.shape
    K = orig_shape[-1]
    N = weight.shape[0]
    x2d = x.reshape(-1, K)
    out = _linear(x2d, weight, bias.reshape(1, N))
    return out.reshape(*orig_shape[:-1], N)
